# baseline (speedup 1.0000x reference)
import jax
import jax.numpy as jnp
from jax import lax
from jax.experimental import pallas as pl
from jax.experimental.pallas import tpu as pltpu
from jax.experimental.pallas import tpu_sc as plsc

NC = 2
NS = 16
NW = NC * NS
L = 16

B = 16384
D = 64
NV = 1000000
TPW = 248
CW = 512
NCHK = TPW * 128 // CW
CB_MAX = (NV + 127) // 128 * 128 - CW
NSTG = 8
ROW_BYTES = D * 4

HID1 = 256
HID2 = 128
BLK = 2048


def _gather_body(ut_hbm, mt_hbm, uid_hbm, mid_hbm, u_out, m_out,
                 idx_all, cidx, cpos, bufa, bufb, stg,
                 sema, semb, osem):
    wid = lax.axis_index("s") * NC + lax.axis_index("c")
    lo128 = wid * (TPW * 128)
    iota16 = lax.iota(jnp.int32, L)

    def one_table(tab_hbm, id_hbm, out_hbm):
        pltpu.sync_copy(id_hbm, idx_all)

        neg1 = jnp.full((L,), -1, jnp.int32)

        def fill(t, _):
            cidx[pl.ds(t * L, L)] = neg1
            return 0

        lax.fori_loop(0, B // L, fill, 0)

        def scan(t, cur):
            vec = idx_all[pl.ds(t * L, L)]
            m = (vec >= lo128) & (vec < lo128 + TPW * 128)
            cnt = plsc.all_reduce_population_count(m)[0]
            plsc.store_compressed(cidx.at[pl.ds(cur, L)], vec, mask=m)
            plsc.store_compressed(cpos.at[pl.ds(cur, L)],
                                  t * L + iota16, mask=m)
            return cur + cnt

        ntot = lax.fori_loop(0, B // L, scan, 0)
        tgrp = (ntot + L - 1) // L

        def chunk_base(c):
            nb = lo128 + c * CW
            return (jnp.minimum(nb, CB_MAX) // 128) * 128, nb

        def issue(c, buf, sem):
            cb, _ = chunk_base(c)
            pltpu.async_copy(tab_hbm.at[:, pl.ds(cb, CW)], buf, sem)

        def process(c, buf, sem, e0):
            cb, nb = chunk_base(c)
            pltpu.make_async_copy(tab_hbm.at[:, pl.ds(0, CW)],
                                  buf, sem).wait()

            def visit(t, e):
                vec = cidx[pl.ds(t * L, L)]
                m = (vec >= nb) & (vec < nb + CW)
                cnt = plsc.all_reduce_population_count(m)[0]

                @pl.when(cnt > 0)
                def _():
                    pvec = cpos[pl.ds(t * L, L)]
                    mi = m.astype(jnp.int32)
                    rank = plsc.cumsum(mi) - mi

                    for lane in range(L):
                        k = e + rank[lane]
                        slot = lax.rem(k, NSTG)

                        @pl.when((mi[lane] == 1) & (k >= NSTG))
                        def _():
                            pltpu.make_async_copy(
                                stg.at[0], out_hbm.at[pl.ds(0, D)],
                                osem).wait()

                        @pl.when(mi[lane] == 1)
                        def _():
                            lv = jnp.full((L,), vec[lane] - cb, jnp.int32)
                            row = stg.at[slot]
                            for j in range(D // L):
                                col = plsc.load_gather(
                                    buf, [iota16 + j * L, lv])
                                row[pl.ds(j * L, L)] = col
                            pltpu.async_copy(
                                row, out_hbm.at[pl.ds(pvec[lane] * D, D)],
                                osem)

                return e + cnt

            return lax.fori_loop(0, tgrp, visit, e0)

        issue(0, bufa, sema)

        def pair(i, e):
            c = 2 * i
            issue(c + 1, bufb, semb)
            e = process(c, bufa, sema, e)
            issue(c + 2, bufa, sema)
            e = process(c + 1, bufb, semb, e)
            return e

        lax.fori_loop(0, NCHK // 2, pair, 0)
        pltpu.make_async_copy(tab_hbm.at[:, pl.ds(0, CW)], bufa, sema).wait()

        def drain(t, _):
            pltpu.make_async_copy(stg.at[0], out_hbm.at[pl.ds(0, D)],
                                  osem).wait()
            return 0

        lax.fori_loop(0, jnp.minimum(ntot, NSTG), drain, 0)

    one_table(ut_hbm, uid_hbm, u_out)
    one_table(mt_hbm, mid_hbm, m_out)


def _sc_gather(ut_t, mt_t, uid, mid):
    mesh = plsc.VectorSubcoreMesh(
        core_axis_name="c", subcore_axis_name="s",
        num_cores=NC, num_subcores=NS)
    fn = pl.kernel(
        _gather_body,
        mesh=mesh,
        compiler_params=pltpu.CompilerParams(needs_layout_passes=False),
        out_type=[
            jax.ShapeDtypeStruct((B * D,), jnp.float32),
            jax.ShapeDtypeStruct((B * D,), jnp.float32),
        ],
        scratch_types=[
            pltpu.VMEM((B,), jnp.int32),
            pltpu.VMEM((B,), jnp.int32),
            pltpu.VMEM((B,), jnp.int32),
            pltpu.VMEM((D, CW), jnp.float32),
            pltpu.VMEM((D, CW), jnp.float32),
            pltpu.VMEM((NSTG, D), jnp.float32),
            pltpu.SemaphoreType.DMA,
            pltpu.SemaphoreType.DMA,
            pltpu.SemaphoreType.DMA,
        ],
    )
    return fn(ut_t, mt_t, uid, mid)


def _mlp_body(u_ref, m_ref, w1t_ref, b1_ref, w2t_ref, b2_ref, w3_ref,
              b3_ref, o_ref):
    x = jnp.concatenate([u_ref[...], m_ref[...]], axis=1)
    h = jnp.dot(x, w1t_ref[...], preferred_element_type=jnp.float32)
    h = jnp.maximum(h + b1_ref[...], 0.0)
    h = jnp.dot(h, w2t_ref[...], preferred_element_type=jnp.float32)
    h = jnp.maximum(h + b2_ref[...], 0.0)
    o_ref[...] = jnp.sum(h * w3_ref[...], axis=1) + b3_ref[0, 0]


def _tc_mlp(u, m, w1t, b1r, w2t, b2r, w3, b3r):
    grid = (B // BLK,)
    return pl.pallas_call(
        _mlp_body,
        grid=grid,
        in_specs=[
            pl.BlockSpec((BLK, D), lambda i: (i, 0)),
            pl.BlockSpec((BLK, D), lambda i: (i, 0)),
            pl.BlockSpec((2 * D, HID1), lambda i: (0, 0)),
            pl.BlockSpec((1, HID1), lambda i: (0, 0)),
            pl.BlockSpec((HID1, HID2), lambda i: (0, 0)),
            pl.BlockSpec((1, HID2), lambda i: (0, 0)),
            pl.BlockSpec((1, HID2), lambda i: (0, 0)),
            pl.BlockSpec((1, 1), lambda i: (0, 0)),
        ],
        out_specs=pl.BlockSpec((BLK,), lambda i: (i,)),
        out_shape=jax.ShapeDtypeStruct((B,), jnp.float32),
    )(u, m, w1t, b1r, w2t, b2r, w3, b3r)


@jax.jit
def kernel(user_ids, movie_ids, user_table, movie_table,
           W1, b1, W2, b2, W3, b3):
    uid = user_ids.astype(jnp.int32)
    mid = movie_ids.astype(jnp.int32)
    uf, mf = _sc_gather(user_table.T, movie_table.T, uid, mid)
    u = uf.reshape(B, D)
    m = mf.reshape(B, D)
    return _tc_mlp(u, m, W1.T, b1.reshape(1, HID1), W2.T,
                   b2.reshape(1, HID2), W3, b3.reshape(1, 1))


# speedup vs baseline: 1.3032x; 1.3032x over previous
import jax
import jax.numpy as jnp
from jax import lax
from jax.experimental import pallas as pl
from jax.experimental.pallas import tpu as pltpu
from jax.experimental.pallas import tpu_sc as plsc

NC = 2
NS = 16
NW = NC * NS
L = 16

B = 16384
D = 64
BPW = B // NW
NGRP = BPW // L
NBUF = 8

HID1 = 256
HID2 = 128
BLK = 2048


def _gather_body(ut_hbm, mt_hbm, uid_hbm, mid_hbm, u_out, m_out,
                 idx_v, rows, *blks_and_sems):
    blks = blks_and_sems[:NBUF]
    sems = blks_and_sems[NBUF:]
    wid = lax.axis_index("s") * NC + lax.axis_index("c")
    base = wid * BPW
    iota16 = lax.iota(jnp.int32, L)

    def one_table(tab_hbm, id_hbm, out_hbm):
        pltpu.sync_copy(id_hbm.at[pl.ds(base, BPW)], idx_v)

        def issue(vec, lane, slot):
            c = (vec[lane] // 128) * 128
            pltpu.async_copy(tab_hbm.at[:, pl.ds(c, 128)],
                             blks[slot], sems[slot])

        def extract(vec, g, lane, slot):
            pltpu.make_async_copy(tab_hbm.at[:, pl.ds(0, 128)],
                                  blks[slot], sems[slot]).wait()
            lv = jnp.full((L,), vec[lane] % 128, jnp.int32)
            for j in range(D // L):
                col = plsc.load_gather(blks[slot], [iota16 + j * L, lv])
                rows[pl.ds((g * L + lane) * D + j * L, L)] = col

        vec0 = idx_v[pl.ds(0, L)]
        for lane in range(NBUF):
            issue(vec0, lane, lane)

        def grp(g, vec):
            nxt = idx_v[pl.ds((g + 1) * L, L)]
            for lane in range(NBUF):
                extract(vec, g, lane, lane)
                issue(vec, lane + NBUF, lane)
            for lane in range(NBUF, L):
                extract(vec, g, lane, lane - NBUF)
                issue(nxt, lane - NBUF, lane - NBUF)
            return nxt

        vlast = lax.fori_loop(0, NGRP - 1, grp, vec0)
        g = NGRP - 1
        for lane in range(NBUF):
            extract(vlast, g, lane, lane)
            issue(vlast, lane + NBUF, lane)
        for lane in range(NBUF, L):
            extract(vlast, g, lane, lane - NBUF)
        pltpu.sync_copy(rows, out_hbm.at[pl.ds(base * D, BPW * D)])

    one_table(ut_hbm, uid_hbm, u_out)
    one_table(mt_hbm, mid_hbm, m_out)


def _sc_gather(ut_t, mt_t, uid, mid):
    mesh = plsc.VectorSubcoreMesh(
        core_axis_name="c", subcore_axis_name="s",
        num_cores=NC, num_subcores=NS)
    fn = pl.kernel(
        _gather_body,
        mesh=mesh,
        compiler_params=pltpu.CompilerParams(needs_layout_passes=False),
        out_type=[
            jax.ShapeDtypeStruct((B * D,), jnp.float32),
            jax.ShapeDtypeStruct((B * D,), jnp.float32),
        ],
        scratch_types=(
            [pltpu.VMEM((BPW,), jnp.int32),
             pltpu.VMEM((BPW * D,), jnp.float32)]
            + [pltpu.VMEM((D, 128), jnp.float32)] * NBUF
            + [pltpu.SemaphoreType.DMA] * NBUF
        ),
    )
    return fn(ut_t, mt_t, uid, mid)


def _mlp_body(u_ref, m_ref, w1t_ref, b1_ref, w2t_ref, b2_ref, w3_ref,
              b3_ref, o_ref):
    x = jnp.concatenate([u_ref[...], m_ref[...]], axis=1)
    h = jnp.dot(x, w1t_ref[...], preferred_element_type=jnp.float32)
    h = jnp.maximum(h + b1_ref[...], 0.0)
    h = jnp.dot(h, w2t_ref[...], preferred_element_type=jnp.float32)
    h = jnp.maximum(h + b2_ref[...], 0.0)
    o_ref[...] = (jnp.sum(h * w3_ref[...], axis=1, keepdims=True)
                  + b3_ref[...])


def _tc_mlp(u, m, w1t, b1r, w2t, b2r, w3, b3r):
    grid = (B // BLK,)
    return pl.pallas_call(
        _mlp_body,
        grid=grid,
        in_specs=[
            pl.BlockSpec((BLK, D), lambda i: (i, 0)),
            pl.BlockSpec((BLK, D), lambda i: (i, 0)),
            pl.BlockSpec((2 * D, HID1), lambda i: (0, 0)),
            pl.BlockSpec((1, HID1), lambda i: (0, 0)),
            pl.BlockSpec((HID1, HID2), lambda i: (0, 0)),
            pl.BlockSpec((1, HID2), lambda i: (0, 0)),
            pl.BlockSpec((1, HID2), lambda i: (0, 0)),
            pl.BlockSpec((1, 1), lambda i: (0, 0)),
        ],
        out_specs=pl.BlockSpec((BLK, 1), lambda i: (i, 0)),
        out_shape=jax.ShapeDtypeStruct((B, 1), jnp.float32),
    )(u, m, w1t, b1r, w2t, b2r, w3, b3r)


@jax.jit
def kernel(user_ids, movie_ids, user_table, movie_table,
           W1, b1, W2, b2, W3, b3):
    uid = user_ids.astype(jnp.int32)
    mid = movie_ids.astype(jnp.int32)
    uf, mf = _sc_gather(user_table.T, movie_table.T, uid, mid)
    u = uf.reshape(B, D)
    m = mf.reshape(B, D)
    out = _tc_mlp(u, m, W1.T, b1.reshape(1, HID1), W2.T,
                  b2.reshape(1, HID2), W3, b3.reshape(1, 1))
    return out[:, 0]


# speedup vs baseline: 1.5319x; 1.1755x over previous
import jax
import jax.numpy as jnp
from jax import lax
from jax.experimental import pallas as pl
from jax.experimental.pallas import tpu as pltpu
from jax.experimental.pallas import tpu_sc as plsc

NC = 2
NS = 16
NW = NC * NS
L = 16

B = 16384
D = 64
NV = 1000000
TPW = 248
CW = 512
NCHK = TPW * 128 // CW
CB_MAX = (NV + 127) // 128 * 128 - CW
NSTG = 8
NSUB = NCHK // 2
ROW_BYTES = D * 4

HID1 = 256
HID2 = 128
BLK = 2048


def _gather_body(ut_hbm, mt_hbm, uid_hbm, mid_hbm, u_out, m_out,
                 idx_all, cent, cent2, bufa, bufb, stg, scnt,
                 sema, semb, osem):
    wid = lax.axis_index("s") * NC + lax.axis_index("c")
    lo128 = wid * (TPW * 128)
    iota16 = lax.iota(jnp.int32, L)

    def one_table(tab_hbm, id_hbm, out_hbm):
        pltpu.sync_copy(id_hbm, idx_all)

        neg1 = jnp.full((L,), -1, jnp.int32)

        def fill(t, _):
            cent[pl.ds(t * L, L)] = neg1
            cent2[pl.ds(t * L, L)] = neg1
            return 0

        lax.fori_loop(0, B // L, fill, 0)

        def scan(t, cur):
            vec = idx_all[pl.ds(t * L, L)]
            rel = vec - lo128
            m = (rel >= 0) & (rel < TPW * 128)
            cnt = plsc.all_reduce_population_count(m)[0]
            packed = rel * 16384 + (t * L + iota16)
            plsc.store_compressed(cent.at[pl.ds(cur, L)], packed, mask=m)
            return cur + cnt

        ntot = lax.fori_loop(0, B // L, scan, 0)
        tgrp = (ntot + L - 1) // L

        for q in range(NSUB + 1):
            scnt[q] = 0
        for q in range(NSUB + 1):
            scnt[NSUB + 1 + q] = 0

        def count(t, _):
            vec = cent[pl.ds(t * L, L)]
            bins = jnp.maximum(jax.lax.shift_right_arithmetic(vec, 24), 0)
            mi = (vec >= 0).astype(jnp.int32)
            for lane in range(L):
                b = bins[lane]

                @pl.when(mi[lane] == 1)
                def _():
                    scnt[b] = scnt[b] + 1

            return 0

        lax.fori_loop(0, tgrp, count, 0)

        acc = 0
        for q in range(NSUB):
            c = scnt[q]
            scnt[NSUB + 1 + q] = acc
            scnt[q] = acc
            acc = acc + c
        scnt[NSUB + 1 + NSUB] = acc

        def redist(t, _):
            vec = cent[pl.ds(t * L, L)]
            bins = jnp.maximum(jax.lax.shift_right_arithmetic(vec, 24), 0)
            mi = (vec >= 0).astype(jnp.int32)
            dvec = jnp.zeros((L,), jnp.int32)
            for lane in range(L):
                b = bins[lane]
                d = scnt[b]

                @pl.when(mi[lane] == 1)
                def _():
                    scnt[b] = d + 1

                dvec = jnp.where(iota16 == lane, d, dvec)
            plsc.store_scatter(cent2, [dvec], vec, mask=vec >= 0)
            return 0

        lax.fori_loop(0, tgrp, redist, 0)

        def chunk_base(c):
            nb = lo128 + c * CW
            return (jnp.minimum(nb, CB_MAX) // 128) * 128, nb

        def issue(c, buf, sem):
            cb, _ = chunk_base(c)
            pltpu.async_copy(tab_hbm.at[:, pl.ds(cb, CW)], buf, sem)

        def process(c, buf, sem, e0):
            cb, nb = chunk_base(c)
            nb_rel = nb - lo128
            q = c // 2
            lo_t = scnt[NSUB + 1 + q] // L
            hi_t = (scnt[NSUB + 2 + q] + L - 1) // L
            pltpu.make_async_copy(tab_hbm.at[:, pl.ds(0, CW)],
                                  buf, sem).wait()

            def visit(t, e):
                vec = cent2[pl.ds(t * L, L)]
                rel = jax.lax.shift_right_arithmetic(vec, 14)
                m = (rel >= nb_rel) & (rel < nb_rel + CW)
                cnt = plsc.all_reduce_population_count(m)[0]

                @pl.when(cnt > 0)
                def _():
                    mi = m.astype(jnp.int32)
                    rank = plsc.cumsum(mi) - mi
                    for lane in range(L):
                        k = e + rank[lane]
                        slot = lax.rem(k, NSTG)

                        @pl.when((mi[lane] == 1) & (k >= NSTG))
                        def _():
                            pltpu.make_async_copy(
                                stg.at[0], out_hbm.at[pl.ds(0, D)],
                                osem).wait()

                        @pl.when(mi[lane] == 1)
                        def _():
                            p = vec[lane]
                            pos = lax.rem(p, 16384)
                            lvs = (jax.lax.shift_right_arithmetic(p, 14)
                                   + lo128 - cb)
                            lv = jnp.full((L,), lvs, jnp.int32)
                            row = stg.at[slot]
                            for j in range(D // L):
                                col = plsc.load_gather(
                                    buf, [iota16 + j * L, lv])
                                row[pl.ds(j * L, L)] = col
                            pltpu.async_copy(
                                row, out_hbm.at[pl.ds(pos * D, D)],
                                osem)

                return e + cnt

            return lax.fori_loop(lo_t, hi_t, visit, e0)

        issue(0, bufa, sema)

        def pair(i, e):
            c = 2 * i
            issue(c + 1, bufb, semb)
            e = process(c, bufa, sema, e)
            issue(c + 2, bufa, sema)
            e = process(c + 1, bufb, semb, e)
            return e

        lax.fori_loop(0, NCHK // 2, pair, 0)
        pltpu.make_async_copy(tab_hbm.at[:, pl.ds(0, CW)], bufa, sema).wait()

        def drain(t, _):
            pltpu.make_async_copy(stg.at[0], out_hbm.at[pl.ds(0, D)],
                                  osem).wait()
            return 0

        lax.fori_loop(0, jnp.minimum(ntot, NSTG), drain, 0)

    one_table(ut_hbm, uid_hbm, u_out)
    one_table(mt_hbm, mid_hbm, m_out)


def _sc_gather(ut_t, mt_t, uid, mid):
    mesh = plsc.VectorSubcoreMesh(
        core_axis_name="c", subcore_axis_name="s",
        num_cores=NC, num_subcores=NS)
    fn = pl.kernel(
        _gather_body,
        mesh=mesh,
        compiler_params=pltpu.CompilerParams(needs_layout_passes=False),
        out_type=[
            jax.ShapeDtypeStruct((B * D,), jnp.float32),
            jax.ShapeDtypeStruct((B * D,), jnp.float32),
        ],
        scratch_types=[
            pltpu.VMEM((B,), jnp.int32),
            pltpu.VMEM((B,), jnp.int32),
            pltpu.VMEM((B,), jnp.int32),
            pltpu.VMEM((D, CW), jnp.float32),
            pltpu.VMEM((D, CW), jnp.float32),
            pltpu.VMEM((NSTG, D), jnp.float32),
            pltpu.SMEM((2 * (NSUB + 1) + 2,), jnp.int32),
            pltpu.SemaphoreType.DMA,
            pltpu.SemaphoreType.DMA,
            pltpu.SemaphoreType.DMA,
        ],
    )
    return fn(ut_t, mt_t, uid, mid)


def _mlp_body(u_ref, m_ref, w1t_ref, b1_ref, w2t_ref, b2_ref, w3_ref,
              b3_ref, o_ref):
    x = jnp.concatenate([u_ref[...], m_ref[...]], axis=1)
    h = jnp.dot(x, w1t_ref[...], preferred_element_type=jnp.float32)
    h = jnp.maximum(h + b1_ref[...], 0.0)
    h = jnp.dot(h, w2t_ref[...], preferred_element_type=jnp.float32)
    h = jnp.maximum(h + b2_ref[...], 0.0)
    o_ref[...] = jnp.sum(h * w3_ref[...], axis=1) + b3_ref[0, 0]


def _tc_mlp(u, m, w1t, b1r, w2t, b2r, w3, b3r):
    grid = (B // BLK,)
    return pl.pallas_call(
        _mlp_body,
        grid=grid,
        in_specs=[
            pl.BlockSpec((BLK, D), lambda i: (i, 0)),
            pl.BlockSpec((BLK, D), lambda i: (i, 0)),
            pl.BlockSpec((2 * D, HID1), lambda i: (0, 0)),
            pl.BlockSpec((1, HID1), lambda i: (0, 0)),
            pl.BlockSpec((HID1, HID2), lambda i: (0, 0)),
            pl.BlockSpec((1, HID2), lambda i: (0, 0)),
            pl.BlockSpec((1, HID2), lambda i: (0, 0)),
            pl.BlockSpec((1, 1), lambda i: (0, 0)),
        ],
        out_specs=pl.BlockSpec((BLK,), lambda i: (i,)),
        out_shape=jax.ShapeDtypeStruct((B,), jnp.float32),
    )(u, m, w1t, b1r, w2t, b2r, w3, b3r)


@jax.jit
def kernel(user_ids, movie_ids, user_table, movie_table,
           W1, b1, W2, b2, W3, b3):
    uid = user_ids.astype(jnp.int32)
    mid = movie_ids.astype(jnp.int32)
    uf, mf = _sc_gather(user_table.T, movie_table.T, uid, mid)
    u = uf.reshape(B, D)
    m = mf.reshape(B, D)
    return _tc_mlp(u, m, W1.T, b1.reshape(1, HID1), W2.T,
                   b2.reshape(1, HID2), W3, b3.reshape(1, 1))
